# Initial kernel scaffold; baseline (speedup 1.0000x reference)
import jax
import jax.numpy as jnp
from jax.experimental import pallas as pl


def kernel(x, conv_w, bn_gamma, bn_beta):
    raise NotImplementedError("write your pallas kernel here")


# speedup vs baseline: 1.1151x; 1.1151x over previous
import jax
import jax.numpy as jnp
from jax.experimental import pallas as pl
from jax.experimental.pallas import tpu as pltpu

_VMEM_LIMIT = 64 * 1024 * 1024


def _tri_diag_weights(conv_w, W, f32):
    Cout, Cin = conv_w.shape[0], conv_w.shape[1]
    w9 = jnp.transpose(conv_w.astype(f32), (2, 3, 1, 0))
    mats = []
    for dy in range(3):
        m = jnp.zeros((W * Cin, W * Cout), f32)
        for dx in range(3):
            shift = jnp.eye(W, k=1 - dx, dtype=f32)
            m = m + jnp.einsum("vw,cd->vcwd", shift, w9[dy, dx]).reshape(
                W * Cin, W * Cout)
        mats.append(m)
    return jnp.stack(mats).astype(jnp.bfloat16)


def kernel(x, conv_w, bn_gamma, bn_beta):
    eps = 1e-5
    N, Cin, H, W = x.shape
    Cout = conv_w.shape[0]
    H2, W2 = H // 2, W // 2
    f32 = jnp.float32
    bf16 = jnp.bfloat16
    WCi, WCo, W2Co = W * Cin, W * Cout, W2 * Cout

    xr = jnp.transpose(x, (0, 2, 3, 1)).reshape(N * H, WCi).astype(bf16)
    B = _tri_diag_weights(conv_w, W, f32)

    def conv_img(x_ref, b_ref):
        zrow = jnp.zeros((1, WCi), bf16)
        band = jnp.concatenate([zrow, x_ref[...], zrow], axis=0)
        acc = jnp.dot(band[0:H], b_ref[0], preferred_element_type=f32)
        acc = acc + jnp.dot(band[1:H + 1], b_ref[1], preferred_element_type=f32)
        acc = acc + jnp.dot(band[2:H + 2], b_ref[2], preferred_element_type=f32)
        return acc

    x_spec = pl.BlockSpec((H, WCi), lambda n: (n, 0))
    b_spec = pl.BlockSpec((3, WCi, WCo), lambda n: (0, 0, 0))

    def stats_kernel(x_ref, b_ref, sum_ref, sq_ref):
        acc = conv_img(x_ref, b_ref)
        sum_ref[...] = jnp.sum(acc, axis=0, keepdims=True).reshape(1, 1, WCo)
        sq_ref[...] = jnp.sum(acc * acc, axis=0, keepdims=True).reshape(1, 1, WCo)

    img_sum, img_sq = pl.pallas_call(
        stats_kernel,
        grid=(N,),
        in_specs=[x_spec, b_spec],
        out_specs=[
            pl.BlockSpec((1, 1, WCo), lambda n: (n, 0, 0)),
            pl.BlockSpec((1, 1, WCo), lambda n: (n, 0, 0)),
        ],
        out_shape=(
            jax.ShapeDtypeStruct((N, 1, WCo), f32),
            jax.ShapeDtypeStruct((N, 1, WCo), f32),
        ),
        compiler_params=pltpu.CompilerParams(
            dimension_semantics=("parallel",),
            vmem_limit_bytes=_VMEM_LIMIT),
    )(xr, B)

    cnt = jnp.float32(N * H * W)
    ch_sum = jnp.sum(img_sum, axis=(0, 1)).reshape(W, Cout).sum(axis=0)
    ch_sq = jnp.sum(img_sq, axis=(0, 1)).reshape(W, Cout).sum(axis=0)
    mean = ch_sum / cnt
    var = jnp.maximum(ch_sq / cnt - mean * mean, 0.0)
    inv = jax.lax.rsqrt(var + eps)
    scale_c = bn_gamma.astype(f32) * inv
    shift_c = bn_beta.astype(f32) - scale_c * mean
    scale = jnp.tile(scale_c, W).reshape(1, WCo)
    shift = jnp.tile(shift_c, W).reshape(1, WCo)

    def fused_kernel(x_ref, b_ref, sc_ref, sh_ref, out_ref, pool_ref, idx_ref):
        acc = conv_img(x_ref, b_ref)
        y = jnp.maximum(acc * sc_ref[...] + sh_ref[...], 0.0)
        out_ref[...] = y

        y4 = y.reshape(H, W2, 2, Cout)
        left, right = y4[:, :, 0, :], y4[:, :, 1, :]
        take_r = right > left
        hv = jnp.where(take_r, right, left)
        hc = take_r.astype(jnp.int32)
        hv2 = hv.reshape(H2, 2, W2, Cout)
        hc2 = hc.reshape(H2, 2, W2, Cout)
        take_b = hv2[:, 1] > hv2[:, 0]
        pv = jnp.where(take_b, hv2[:, 1], hv2[:, 0])
        pc = jnp.where(take_b, hc2[:, 1], hc2[:, 0])
        pr = take_b.astype(jnp.int32)

        ii = jax.lax.broadcasted_iota(jnp.int32, (H2, W2, Cout), 0)
        jj = jax.lax.broadcasted_iota(jnp.int32, (H2, W2, Cout), 1)
        flat = (2 * ii + pr) * W + (2 * jj + pc)

        pool_ref[...] = pv.reshape(H2, W2Co)
        idx_ref[...] = flat.reshape(H2, W2Co)

    out2d, pool2d, idx2d = pl.pallas_call(
        fused_kernel,
        grid=(N,),
        in_specs=[x_spec, b_spec,
                  pl.BlockSpec((1, WCo), lambda n: (0, 0)),
                  pl.BlockSpec((1, WCo), lambda n: (0, 0))],
        out_specs=[
            pl.BlockSpec((H, WCo), lambda n: (n, 0)),
            pl.BlockSpec((H2, W2Co), lambda n: (n, 0)),
            pl.BlockSpec((H2, W2Co), lambda n: (n, 0)),
        ],
        out_shape=(
            jax.ShapeDtypeStruct((N * H, WCo), f32),
            jax.ShapeDtypeStruct((N * H2, W2Co), f32),
            jax.ShapeDtypeStruct((N * H2, W2Co), jnp.int32),
        ),
        compiler_params=pltpu.CompilerParams(
            dimension_semantics=("parallel",),
            vmem_limit_bytes=_VMEM_LIMIT),
    )(xr, B, scale, shift)

    out = jnp.transpose(out2d.reshape(N, H, W, Cout), (0, 3, 1, 2))
    logits = jnp.transpose(pool2d.reshape(N, H2, W2, Cout), (0, 3, 1, 2))
    indices = jnp.transpose(idx2d.reshape(N, H2, W2, Cout), (0, 3, 1, 2))
    return logits, indices, out


# speedup vs baseline: 2.7711x; 2.4850x over previous
import jax
import jax.numpy as jnp
from jax.experimental import pallas as pl
from jax.experimental.pallas import tpu as pltpu

_VMEM_LIMIT = 64 * 1024 * 1024


def _tri_diag_weights(conv_w, W, f32):
    Cout, Cin = conv_w.shape[0], conv_w.shape[1]
    W2 = W // 2
    w9 = jnp.transpose(conv_w.astype(f32), (2, 3, 1, 0))
    mats = []
    for dy in range(3):
        m = jnp.zeros((W * Cin, W * Cout), f32)
        for dx in range(3):
            shift = jnp.eye(W, k=1 - dx, dtype=f32)
            m = m + jnp.einsum("vw,cd->vcwd", shift, w9[dy, dx]).reshape(
                W * Cin, W * Cout)
        m = m.reshape(W * Cin, W2, 2, Cout).transpose(0, 2, 1, 3).reshape(
            W * Cin, W * Cout)
        mats.append(m)
    return jnp.stack(mats).astype(jnp.bfloat16)


def kernel(x, conv_w, bn_gamma, bn_beta):
    eps = 1e-5
    N, Cin, H, W = x.shape
    Cout = conv_w.shape[0]
    H2, W2 = H // 2, W // 2
    f32 = jnp.float32
    bf16 = jnp.bfloat16
    WCi, WCo, W2Co = W * Cin, W * Cout, W2 * Cout

    xr = jnp.transpose(x, (0, 2, 3, 1)).reshape(N * H, WCi).astype(bf16)
    B = _tri_diag_weights(conv_w, W, f32)

    def conv_img(x_ref, b_ref):
        zrow = jnp.zeros((1, WCi), bf16)
        band = jnp.concatenate([zrow, x_ref[...], zrow], axis=0)
        acc = jnp.dot(band[0:H], b_ref[0], preferred_element_type=f32)
        acc = acc + jnp.dot(band[1:H + 1], b_ref[1], preferred_element_type=f32)
        acc = acc + jnp.dot(band[2:H + 2], b_ref[2], preferred_element_type=f32)
        return acc

    x_spec = pl.BlockSpec((H, WCi), lambda n: (n, 0))
    b_spec = pl.BlockSpec((3, WCi, WCo), lambda n: (0, 0, 0))

    def stats_kernel(x_ref, b_ref, sum_ref, sq_ref):
        acc = conv_img(x_ref, b_ref)
        sum_ref[...] = jnp.sum(acc, axis=0, keepdims=True).reshape(1, 1, WCo)
        sq_ref[...] = jnp.sum(acc * acc, axis=0, keepdims=True).reshape(1, 1, WCo)

    img_sum, img_sq = pl.pallas_call(
        stats_kernel,
        grid=(N,),
        in_specs=[x_spec, b_spec],
        out_specs=[
            pl.BlockSpec((1, 1, WCo), lambda n: (n, 0, 0)),
            pl.BlockSpec((1, 1, WCo), lambda n: (n, 0, 0)),
        ],
        out_shape=(
            jax.ShapeDtypeStruct((N, 1, WCo), f32),
            jax.ShapeDtypeStruct((N, 1, WCo), f32),
        ),
        compiler_params=pltpu.CompilerParams(
            dimension_semantics=("parallel",),
            vmem_limit_bytes=_VMEM_LIMIT),
    )(xr, B)

    cnt = jnp.float32(N * H * W)
    ch_sum = jnp.sum(img_sum, axis=(0, 1)).reshape(W, Cout).sum(axis=0)
    ch_sq = jnp.sum(img_sq, axis=(0, 1)).reshape(W, Cout).sum(axis=0)
    mean = ch_sum / cnt
    var = jnp.maximum(ch_sq / cnt - mean * mean, 0.0)
    inv = jax.lax.rsqrt(var + eps)
    scale_c = bn_gamma.astype(f32) * inv
    shift_c = bn_beta.astype(f32) - scale_c * mean
    scale = jnp.tile(scale_c, W).reshape(1, WCo)
    shift = jnp.tile(shift_c, W).reshape(1, WCo)

    def fused_kernel(x_ref, b_ref, sc_ref, sh_ref, out_ref, pool_ref, idx_ref):
        acc = conv_img(x_ref, b_ref)
        y = jnp.maximum(acc * sc_ref[...] + sh_ref[...], 0.0)
        out_ref[...] = y

        left, right = y[:, :W2Co], y[:, W2Co:]
        take_r = right > left
        hv = jnp.where(take_r, right, left)
        hc = take_r.astype(jnp.int32)
        hv2 = hv.reshape(H2, 2, W2Co)
        hc2 = hc.reshape(H2, 2, W2Co)
        top_v, bot_v = hv2[:, 0], hv2[:, 1]
        take_b = bot_v > top_v
        pv = jnp.where(take_b, bot_v, top_v)
        pc = jnp.where(take_b, hc2[:, 1], hc2[:, 0])
        pr = take_b.astype(jnp.int32)

        ii = jax.lax.broadcasted_iota(jnp.int32, (H2, W2Co), 0)
        ll = jax.lax.broadcasted_iota(jnp.int32, (H2, W2Co), 1)
        jj = ll // Cout
        flat = (2 * ii + pr) * W + (2 * jj + pc)

        pool_ref[...] = pv
        idx_ref[...] = flat

    out2d, pool2d, idx2d = pl.pallas_call(
        fused_kernel,
        grid=(N,),
        in_specs=[x_spec, b_spec,
                  pl.BlockSpec((1, WCo), lambda n: (0, 0)),
                  pl.BlockSpec((1, WCo), lambda n: (0, 0))],
        out_specs=[
            pl.BlockSpec((H, WCo), lambda n: (n, 0)),
            pl.BlockSpec((H2, W2Co), lambda n: (n, 0)),
            pl.BlockSpec((H2, W2Co), lambda n: (n, 0)),
        ],
        out_shape=(
            jax.ShapeDtypeStruct((N * H, WCo), f32),
            jax.ShapeDtypeStruct((N * H2, W2Co), f32),
            jax.ShapeDtypeStruct((N * H2, W2Co), jnp.int32),
        ),
        compiler_params=pltpu.CompilerParams(
            dimension_semantics=("parallel",),
            vmem_limit_bytes=_VMEM_LIMIT),
    )(xr, B, scale, shift)

    out = jnp.transpose(out2d.reshape(N, H, 2, W2, Cout),
                        (0, 4, 1, 3, 2)).reshape(N, Cout, H, W)
    logits = jnp.transpose(pool2d.reshape(N, H2, W2, Cout), (0, 3, 1, 2))
    indices = jnp.transpose(idx2d.reshape(N, H2, W2, Cout), (0, 3, 1, 2))
    return logits, indices, out


# speedup vs baseline: 3.2583x; 1.1758x over previous
import jax
import jax.numpy as jnp
from jax.experimental import pallas as pl
from jax.experimental.pallas import tpu as pltpu

_VMEM_LIMIT = 64 * 1024 * 1024


def _tri_diag_weights(conv_w, W, f32):
    Cout, Cin = conv_w.shape[0], conv_w.shape[1]
    W2 = W // 2
    w9 = jnp.transpose(conv_w.astype(f32), (2, 3, 1, 0))
    mats = []
    for dy in range(3):
        m = jnp.zeros((W * Cin, W * Cout), f32)
        for dx in range(3):
            shift = jnp.eye(W, k=1 - dx, dtype=f32)
            m = m + jnp.einsum("vw,cd->vcwd", shift, w9[dy, dx]).reshape(
                W * Cin, W * Cout)
        m = m.reshape(W * Cin, W2, 2, Cout).transpose(0, 2, 1, 3).reshape(
            W * Cin, W * Cout)
        mats.append(m)
    return jnp.stack(mats).astype(jnp.bfloat16)


def kernel(x, conv_w, bn_gamma, bn_beta):
    eps = 1e-5
    N, Cin, H, W = x.shape
    Cout = conv_w.shape[0]
    H2, W2 = H // 2, W // 2
    f32 = jnp.float32
    bf16 = jnp.bfloat16
    WCi, WCo, W2Co = W * Cin, W * Cout, W2 * Cout

    xr = jnp.transpose(x, (0, 2, 3, 1)).reshape(N * H, WCi).astype(bf16)
    B = _tri_diag_weights(conv_w, W, f32)

    IPB = 8 if N % 8 == 0 else 1
    G = N // IPB

    def conv_img(ximg, b_ref):
        zrow = jnp.zeros((1, WCi), bf16)
        band = jnp.concatenate([zrow, ximg, zrow], axis=0)
        acc = jnp.dot(band[0:H], b_ref[0], preferred_element_type=f32)
        acc = acc + jnp.dot(band[1:H + 1], b_ref[1], preferred_element_type=f32)
        acc = acc + jnp.dot(band[2:H + 2], b_ref[2], preferred_element_type=f32)
        return acc

    x_spec = pl.BlockSpec((IPB * H, WCi), lambda n: (n, 0))
    b_spec = pl.BlockSpec((3, WCi, WCo), lambda n: (0, 0, 0))

    def stats_kernel(x_ref, b_ref, sum_ref, sq_ref):
        s = jnp.zeros((1, WCo), f32)
        q = jnp.zeros((1, WCo), f32)
        for i in range(IPB):
            acc = conv_img(x_ref[i * H:(i + 1) * H], b_ref)
            s = s + jnp.sum(acc, axis=0, keepdims=True)
            q = q + jnp.sum(acc * acc, axis=0, keepdims=True)
        sum_ref[...] = s.reshape(1, 1, WCo)
        sq_ref[...] = q.reshape(1, 1, WCo)

    img_sum, img_sq = pl.pallas_call(
        stats_kernel,
        grid=(G,),
        in_specs=[x_spec, b_spec],
        out_specs=[
            pl.BlockSpec((1, 1, WCo), lambda n: (n, 0, 0)),
            pl.BlockSpec((1, 1, WCo), lambda n: (n, 0, 0)),
        ],
        out_shape=(
            jax.ShapeDtypeStruct((G, 1, WCo), f32),
            jax.ShapeDtypeStruct((G, 1, WCo), f32),
        ),
        compiler_params=pltpu.CompilerParams(
            dimension_semantics=("parallel",),
            vmem_limit_bytes=_VMEM_LIMIT),
    )(xr, B)

    cnt = jnp.float32(N * H * W)
    ch_sum = jnp.sum(img_sum, axis=(0, 1)).reshape(W, Cout).sum(axis=0)
    ch_sq = jnp.sum(img_sq, axis=(0, 1)).reshape(W, Cout).sum(axis=0)
    mean = ch_sum / cnt
    var = jnp.maximum(ch_sq / cnt - mean * mean, 0.0)
    inv = jax.lax.rsqrt(var + eps)
    scale_c = bn_gamma.astype(f32) * inv
    shift_c = bn_beta.astype(f32) - scale_c * mean
    scale = jnp.tile(scale_c, W).reshape(1, WCo)
    shift = jnp.tile(shift_c, W).reshape(1, WCo)

    def fused_kernel(x_ref, b_ref, sc_ref, sh_ref, out_ref, pool_ref, idx_ref):
        ii = jax.lax.broadcasted_iota(jnp.int32, (H2, W2Co), 0)
        ll = jax.lax.broadcasted_iota(jnp.int32, (H2, W2Co), 1)
        jj = ll // Cout
        for i in range(IPB):
            acc = conv_img(x_ref[i * H:(i + 1) * H], b_ref)
            y = jnp.maximum(acc * sc_ref[...] + sh_ref[...], 0.0)
            out_ref[i * H:(i + 1) * H, :] = y

            left, right = y[:, :W2Co], y[:, W2Co:]
            take_r = right > left
            hv = jnp.where(take_r, right, left)
            hc = take_r.astype(jnp.int32)
            hv2 = hv.reshape(H2, 2, W2Co)
            hc2 = hc.reshape(H2, 2, W2Co)
            top_v, bot_v = hv2[:, 0], hv2[:, 1]
            take_b = bot_v > top_v
            pv = jnp.where(take_b, bot_v, top_v)
            pc = jnp.where(take_b, hc2[:, 1], hc2[:, 0])
            pr = take_b.astype(jnp.int32)

            flat = (2 * ii + pr) * W + (2 * jj + pc)
            pool_ref[i * H2:(i + 1) * H2, :] = pv
            idx_ref[i * H2:(i + 1) * H2, :] = flat

    out2d, pool2d, idx2d = pl.pallas_call(
        fused_kernel,
        grid=(G,),
        in_specs=[x_spec, b_spec,
                  pl.BlockSpec((1, WCo), lambda n: (0, 0)),
                  pl.BlockSpec((1, WCo), lambda n: (0, 0))],
        out_specs=[
            pl.BlockSpec((IPB * H, WCo), lambda n: (n, 0)),
            pl.BlockSpec((IPB * H2, W2Co), lambda n: (n, 0)),
            pl.BlockSpec((IPB * H2, W2Co), lambda n: (n, 0)),
        ],
        out_shape=(
            jax.ShapeDtypeStruct((N * H, WCo), f32),
            jax.ShapeDtypeStruct((N * H2, W2Co), f32),
            jax.ShapeDtypeStruct((N * H2, W2Co), jnp.int32),
        ),
        compiler_params=pltpu.CompilerParams(
            dimension_semantics=("parallel",),
            vmem_limit_bytes=_VMEM_LIMIT),
    )(xr, B, scale, shift)

    out = jnp.transpose(out2d.reshape(N, H, 2, W2, Cout),
                        (0, 4, 1, 3, 2)).reshape(N, Cout, H, W)
    logits = jnp.transpose(pool2d.reshape(N, H2, W2, Cout), (0, 3, 1, 2))
    indices = jnp.transpose(idx2d.reshape(N, H2, W2, Cout), (0, 3, 1, 2))
    return logits, indices, out
